# baseline (speedup 1.0000x reference)
import jax
import jax.numpy as jnp
from jax import lax
from jax.experimental import pallas as pl
from jax.experimental.pallas import tpu as pltpu

_N_VOCAB = 1000000
_N_EMBED = 64
_BATCH = 16384
_TC_BLK = 512


def _tc_gather_kernel(idx_smem, tbl_hbm, out_vmem, sem):
    g = pl.program_id(0)

    def fire(j, _):
        i = idx_smem[g * _TC_BLK + j]
        pltpu.make_async_copy(
            tbl_hbm.at[pl.ds(i, 1)], out_vmem.at[pl.ds(j, 1)], sem
        ).start()
        return ()

    lax.fori_loop(0, _TC_BLK, fire, (), unroll=16)
    pltpu.make_async_copy(
        tbl_hbm.at[pl.ds(0, _TC_BLK)], out_vmem, sem
    ).wait()


@jax.jit
def kernel(input_words, in_embed_weight):
    grid_spec = pltpu.PrefetchScalarGridSpec(
        num_scalar_prefetch=1,
        grid=(_BATCH // _TC_BLK,),
        in_specs=[pl.BlockSpec(memory_space=pl.ANY)],
        out_specs=pl.BlockSpec((_TC_BLK, _N_EMBED), lambda g, idx: (g, 0)),
        scratch_shapes=[pltpu.SemaphoreType.DMA],
    )
    out = pl.pallas_call(
        _tc_gather_kernel,
        grid_spec=grid_spec,
        out_shape=jax.ShapeDtypeStruct((_BATCH, _N_EMBED), jnp.float32),
    )(input_words, in_embed_weight)
    return out


# speedup vs baseline: 1.1745x; 1.1745x over previous
import functools

import jax
import jax.numpy as jnp
from jax import lax
from jax.experimental import pallas as pl
from jax.experimental.pallas import tpu as pltpu
from jax.experimental.pallas import tpu_sc as plsc

_N_VOCAB = 1000000
_N_EMBED = 64
_BATCH = 16384

_info = plsc.get_sparse_core_info()
_NC = _info.num_cores
_NS = _info.num_subcores
_NW = _NC * _NS
_B_PER_W = _BATCH // _NW
_K = 16
_NBATCH = _B_PER_W // _K


def _gather_kernel(tbl_hbm, idx_hbm, out_hbm, idx_v, rows_v, sem):
    wid = lax.axis_index("s") * _NC + lax.axis_index("c")
    base = wid * _B_PER_W
    pltpu.sync_copy(idx_hbm.at[wid], idx_v)

    def batch_body(b, _):
        vblk = idx_v[pl.ds(b * _K, _K)]
        for l in range(_K):
            i = vblk[l]
            pltpu.async_copy(
                tbl_hbm.at[pl.ds(i, 1), :],
                rows_v.at[pl.ds(b * _K + l, 1), :],
                sem,
            )
        return ()

    lax.fori_loop(0, _NBATCH, batch_body, (), unroll=False)

    def drain_body(b, _):
        pltpu.make_async_copy(
            tbl_hbm.at[pl.ds(0, 1), :], rows_v.at[pl.ds(0, 1), :], sem
        ).wait()
        return ()

    lax.fori_loop(0, _B_PER_W, drain_body, (), unroll=False)
    pltpu.sync_copy(rows_v, out_hbm.at[pl.ds(base, _B_PER_W)])


@jax.jit
def kernel(input_words, in_embed_weight):
    idx = input_words.reshape(_NW, _B_PER_W)
    tbl = jnp.copy(in_embed_weight)
    mesh = plsc.VectorSubcoreMesh(core_axis_name="c", subcore_axis_name="s")
    out = pl.kernel(
        _gather_kernel,
        mesh=mesh,
        out_type=jax.ShapeDtypeStruct((_BATCH, _N_EMBED), jnp.float32),
        scratch_types=[
            pltpu.VMEM((_B_PER_W,), jnp.int32),
            pltpu.VMEM((_B_PER_W, _N_EMBED), jnp.float32),
            pltpu.SemaphoreType.DMA,
        ],
    )(tbl, idx)
    return out


# speedup vs baseline: 1.7557x; 1.4948x over previous
import functools

import jax
import jax.numpy as jnp
from jax import lax
from jax.experimental import pallas as pl
from jax.experimental.pallas import tpu as pltpu
from jax.experimental.pallas import tpu_sc as plsc

_N_VOCAB = 1000000
_N_EMBED = 64
_BATCH = 16384

_info = plsc.get_sparse_core_info()
_NC = _info.num_cores
_NS = _info.num_subcores
_NW = _NC * _NS
_B_PER_W = _BATCH // _NW
_K = 16
_NBATCH = _B_PER_W // _K


def _gather_kernel(tbl_hbm, idx_hbm, out_hbm, idx_v, rows_v, sem):
    wid = lax.axis_index("s") * _NC + lax.axis_index("c")
    base = wid * _B_PER_W
    pltpu.sync_copy(idx_hbm.at[wid], idx_v)

    def batch_body(b, _):
        vblk = idx_v[pl.ds(b * _K, _K)]
        for l in range(_K):
            i = vblk[l]
            pltpu.async_copy(
                tbl_hbm.at[0, pl.ds(i, 1), :],
                rows_v.at[pl.ds(b * _K + l, 1), :],
                sem,
            )
        return ()

    lax.fori_loop(0, _NBATCH, batch_body, (), unroll=False)

    def drain_body(b, _):
        pltpu.make_async_copy(
            tbl_hbm.at[0, pl.ds(0, 1), :], rows_v.at[pl.ds(0, 1), :], sem
        ).wait()
        return ()

    lax.fori_loop(0, _B_PER_W, drain_body, (), unroll=False)
    pltpu.sync_copy(rows_v, out_hbm.at[pl.ds(base, _B_PER_W)])


@jax.jit
def kernel(input_words, in_embed_weight):
    idx = input_words.reshape(_NW, _B_PER_W)
    tbl = in_embed_weight.reshape(1, _N_VOCAB, _N_EMBED)
    mesh = plsc.VectorSubcoreMesh(core_axis_name="c", subcore_axis_name="s")
    out = pl.kernel(
        _gather_kernel,
        mesh=mesh,
        out_type=jax.ShapeDtypeStruct((_BATCH, _N_EMBED), jnp.float32),
        scratch_types=[
            pltpu.VMEM((_B_PER_W,), jnp.int32),
            pltpu.VMEM((_B_PER_W, _N_EMBED), jnp.float32),
            pltpu.SemaphoreType.DMA,
        ],
    )(tbl, idx)
    return out
